# baseline (speedup 1.0000x reference)
import functools

import jax
import jax.numpy as jnp
from jax import lax
from jax.experimental import pallas as pl
from jax.experimental.pallas import tpu as pltpu
from jax.experimental.pallas import tpu_sc as plsc

CHUNK = 64
GRP = 5


def _make_gather(n_total, d):
    info = plsc.get_sparse_core_info()
    nc, ns = info.num_cores, info.num_subcores
    nw = nc * ns
    per_w = n_total // nw
    n_chunks = per_w // CHUNK
    n_grp = n_chunks // GRP
    assert per_w % CHUNK == 0 and n_chunks % GRP == 0 and n_grp % 2 == 0

    mesh = plsc.VectorSubcoreMesh(core_axis_name="c", subcore_axis_name="s")

    @functools.partial(
        pl.kernel,
        mesh=mesh,
        out_type=jax.ShapeDtypeStruct((nw, n_grp, GRP * CHUNK, d), jnp.float32),
        scratch_types=[
            pltpu.VMEM((n_chunks, CHUNK), jnp.int32),
            pltpu.VMEM((2, GRP * CHUNK, d), jnp.float32),
            pltpu.SemaphoreType.DMA((2,)),
            pltpu.SemaphoreType.DMA((2,)),
        ],
    )
    def gather_kernel(idx_hbm, table_hbm, out_hbm, idx_v, rows_v, gsem, ssem):
        wid = lax.axis_index("s") * nc + lax.axis_index("c")
        pltpu.sync_copy(idx_hbm.at[wid], idx_v)

        def fire_gathers(grp, p):
            for g in range(GRP):
                pltpu.make_async_copy(
                    table_hbm.at[idx_v.at[grp * GRP + g]],
                    rows_v.at[p, pl.ds(g * CHUNK, CHUNK)],
                    gsem.at[p],
                ).start()

        def wait_gathers(p):
            for g in range(GRP):
                pltpu.make_async_copy(
                    table_hbm.at[idx_v.at[0]],
                    rows_v.at[p, pl.ds(g * CHUNK, CHUNK)],
                    gsem.at[p],
                ).wait()

        def fire_store(grp, p):
            pltpu.make_async_copy(
                rows_v.at[p], out_hbm.at[wid, grp], ssem.at[p]
            ).start()

        def wait_store(p):
            pltpu.make_async_copy(
                rows_v.at[p], out_hbm.at[wid, 0], ssem.at[p]
            ).wait()

        fire_gathers(0, 0)

        def body(gg, _):
            for p in (1, 0):
                grp = 2 * gg + (1 if p == 1 else 2)

                @pl.when(grp < n_grp)
                def _():
                    @pl.when(grp >= 2)
                    def _():
                        wait_store(p)
                    fire_gathers(grp, p)

                q = 1 - p
                wait_gathers(q)
                fire_store(grp - 1, q)

            return 0

        lax.fori_loop(0, n_grp // 2, body, 0)

        wait_store(0)
        wait_store(1)

    return gather_kernel, nw, n_grp


def kernel(indices, table):
    bsz, seq = indices.shape
    _, d = table.shape
    n_total = bsz * seq

    gather_kernel, nw, n_grp = _make_gather(n_total, d)
    idx = indices.astype(jnp.int32).reshape(nw, n_grp * GRP, CHUNK)
    out = gather_kernel(idx, table)
    emb = out.reshape(bsz, seq, d)
    seq_lengths = jnp.full((bsz,), seq, dtype=jnp.int32)
    return (emb, seq_lengths)


# speedup vs baseline: 1.0177x; 1.0177x over previous
import functools

import jax
import jax.numpy as jnp
from jax import lax
from jax.experimental import pallas as pl
from jax.experimental.pallas import tpu as pltpu
from jax.experimental.pallas import tpu_sc as plsc

CHUNK = 64
NBUF = 10
LAG = 1


def _make_gather(n_total, d):
    info = plsc.get_sparse_core_info()
    nc, ns = info.num_cores, info.num_subcores
    nw = nc * ns
    per_w = n_total // nw
    n_chunks = per_w // CHUNK
    n_groups = n_chunks // NBUF
    assert per_w % CHUNK == 0 and n_chunks % NBUF == 0

    mesh = plsc.VectorSubcoreMesh(core_axis_name="c", subcore_axis_name="s")

    @functools.partial(
        pl.kernel,
        mesh=mesh,
        out_type=jax.ShapeDtypeStruct((nw, n_chunks, CHUNK, d), jnp.float32),
        scratch_types=[
            pltpu.VMEM((n_chunks, CHUNK), jnp.int32),
            pltpu.VMEM((NBUF, CHUNK, d), jnp.float32),
            pltpu.SemaphoreType.DMA((NBUF,)),
            pltpu.SemaphoreType.DMA((NBUF,)),
        ],
    )
    def gather_kernel(idx_hbm, table_hbm, out_hbm, idx_v, rows_v, gsem, ssem):
        wid = lax.axis_index("s") * nc + lax.axis_index("c")
        pltpu.sync_copy(idx_hbm.at[wid], idx_v)

        def fire_gather(chunk, b):
            pltpu.make_async_copy(
                table_hbm.at[idx_v.at[chunk]], rows_v.at[b], gsem.at[b]
            ).start()

        def wait_gather(b):
            pltpu.make_async_copy(
                table_hbm.at[idx_v.at[0]], rows_v.at[b], gsem.at[b]
            ).wait()

        def fire_store(chunk, b):
            pltpu.make_async_copy(
                rows_v.at[b], out_hbm.at[wid, chunk], ssem.at[b]
            ).start()

        def wait_store(b):
            pltpu.make_async_copy(
                rows_v.at[b], out_hbm.at[wid, 0], ssem.at[b]
            ).wait()

        for b in range(NBUF):
            fire_gather(b, b)

        def body(g, _):
            for b in range(NBUF):
                j = g * NBUF + b
                wait_gather(b)
                fire_store(j, b)
                bfill = (b - LAG) % NBUF

                @pl.when((j >= LAG) & (j + NBUF - LAG < n_chunks))
                def _():
                    wait_store(bfill)
                    fire_gather(j + NBUF - LAG, bfill)

            return 0

        lax.fori_loop(0, n_chunks // NBUF, body, 0)

        for b in range(NBUF):
            wait_store(b)

    return gather_kernel, nw, n_chunks


def kernel(indices, table):
    bsz, seq = indices.shape
    _, d = table.shape
    n_total = bsz * seq

    gather_kernel, nw, n_chunks = _make_gather(n_total, d)
    idx = indices.astype(jnp.int32).reshape(nw, n_chunks, CHUNK)
    out = gather_kernel(idx, table)
    emb = out.reshape(bsz, seq, d)
    seq_lengths = jnp.full((bsz,), seq, dtype=jnp.int32)
    return (emb, seq_lengths)
